# Initial kernel scaffold; baseline (speedup 1.0000x reference)
import jax
import jax.numpy as jnp
from jax.experimental import pallas as pl


def kernel(x, edge_index, W_F1, asrc_F1, adst_F1, b_F1, W_F2, asrc_F2, adst_F2, b_F2, W_G1, asrc_G1, adst_G1, b_G1, W_G2, asrc_G2, adst_G2, b_G2):
    raise NotImplementedError("write your pallas kernel here")


# speedup vs baseline: 33.9373x; 33.9373x over previous
import functools

import jax
import jax.numpy as jnp
from jax import lax
from jax.experimental import pallas as pl
from jax.experimental.pallas import tpu as pltpu
from jax.experimental.pallas import tpu_sc as plsc

N = 50000
E = 1600000
D = 16

NPAD_DENSE = 50176
BM_DENSE = 512
NPAD_SC = 51200
ROWS_PER_TILE = NPAD_SC // 16
B = 128
E_TOT = E + N
CHUNKS_PER_TILE = -(-E_TOT // (16 * B))
E_PAD = 16 * B * CHUNKS_PER_TILE
DENP_ROWS = NPAD_SC // 4
DENP_PER_TILE = DENP_ROWS // 16
BM_EPI = 400

_i32 = jnp.int32
_f32 = jnp.float32


def _dense_body(x_ref, w_ref, p_ref, srcf_ref, srcg_ref, ad_ref):
    x = x_ref[...]
    h = jnp.dot(x, w_ref[...], preferred_element_type=_f32)
    a = jnp.dot(h, p_ref[...], preferred_element_type=_f32)
    zpad = jnp.zeros((x.shape[0], 14), _f32)
    srcf_ref[...] = jnp.concatenate([h[:, 0:32], a[:, 0:2], zpad], axis=1)
    srcg_ref[...] = jnp.concatenate([h[:, 32:64], a[:, 2:4], zpad], axis=1)
    ad_ref[...] = jnp.concatenate([a[:, 4:8], jnp.zeros((x.shape[0], 4), _f32)],
                                  axis=1)


def _dense_tables(x_pad, w_all, p_all):
    grid = NPAD_DENSE // BM_DENSE
    return pl.pallas_call(
        _dense_body,
        grid=(grid,),
        in_specs=[
            pl.BlockSpec((BM_DENSE, 32), lambda i: (i, 0)),
            pl.BlockSpec((32, 64), lambda i: (0, 0)),
            pl.BlockSpec((64, 8), lambda i: (0, 0)),
        ],
        out_specs=[
            pl.BlockSpec((BM_DENSE, 48), lambda i: (i, 0)),
            pl.BlockSpec((BM_DENSE, 48), lambda i: (i, 0)),
            pl.BlockSpec((BM_DENSE, 8), lambda i: (i, 0)),
        ],
        out_shape=[
            jax.ShapeDtypeStruct((NPAD_DENSE, 48), _f32),
            jax.ShapeDtypeStruct((NPAD_DENSE, 48), _f32),
            jax.ShapeDtypeStruct((NPAD_DENSE, 8), _f32),
        ],
    )(x_pad, w_all, p_all)


def _zero_bufs(numbuf, wbuf8):
    zv = jnp.zeros((16,), _f32)
    cols = lax.iota(_i32, 16)

    @pl.loop(0, B)
    def _(e):
        row = jnp.full((16,), e, _i32)
        plsc.store_scatter(numbuf, [row, cols], zv)
        plsc.store_scatter(numbuf, [row, cols + 16], zv)

    @pl.loop(0, B // 2)
    def _(e):
        row = e * 2 + (cols >> 3)
        plsc.store_scatter(wbuf8, [row, cols & 7], zv)


def _main_loop(tbl_hbm, ad_hbm, srci_hbm, dsti_hbm, adc,
               sbuf, dbuf, dbuf4, srcbuf, adbuf, numbuf, wbuf8,
               num_acc, denp_acc, s):
    cols = lax.iota(_i32, 16)
    zv = jnp.zeros((16,), _f32)

    @pl.loop(0, CHUNKS_PER_TILE)
    def _(g):
        base = (s * CHUNKS_PER_TILE + g) * B
        pltpu.sync_copy(srci_hbm.at[pl.ds(base, B)], sbuf)
        pltpu.sync_copy(dsti_hbm.at[pl.ds(base, B)], dbuf)
        pltpu.sync_copy(tbl_hbm.at[sbuf], srcbuf)
        pltpu.sync_copy(ad_hbm.at[dbuf], adbuf)

        for j in range(B // 16):
            rows = cols + j * 16
            as0 = plsc.load_gather(srcbuf, [rows, jnp.full((16,), 32, _i32)])
            as1 = plsc.load_gather(srcbuf, [rows, jnp.full((16,), 33, _i32)])
            ad0 = plsc.load_gather(adbuf, [rows, adc])
            ad1 = plsc.load_gather(adbuf, [rows, adc + 1])
            al0 = as0 + ad0
            al1 = as1 + ad1
            w0 = jnp.exp(jnp.where(al0 >= 0.0, al0, al0 * 0.2))
            w1 = jnp.exp(jnp.where(al1 >= 0.0, al1, al1 * 0.2))
            d = plsc.load_gather(dbuf, [rows])
            plsc.store_scatter(dbuf4, [rows], d >> 2)
            for k in range(8):
                plsc.store_scatter(wbuf8, [rows, jnp.full((16,), k, _i32)], zv)
            dcol = (d & 3) * 2
            plsc.store_scatter(wbuf8, [rows, dcol], w0)
            plsc.store_scatter(wbuf8, [rows, dcol + 1], w1)
            for k in range(16):
                ck = jnp.full((16,), k, _i32)
                hk = plsc.load_gather(srcbuf, [rows, ck])
                plsc.store_scatter(numbuf, [rows, ck], hk * w0)
            for k in range(16, 32):
                ck = jnp.full((16,), k, _i32)
                hk = plsc.load_gather(srcbuf, [rows, ck])
                plsc.store_scatter(numbuf, [rows, ck], hk * w1)

        pltpu.sync_copy(numbuf, num_acc.at[dbuf], add=True)
        pltpu.sync_copy(wbuf8, denp_acc.at[dbuf4], add=True)


def _edge_kernel(srcf, srcg, ad, srci, dsti,
                 numf, denpf, numg, denpg,
                 sbuf, dbuf, dbuf4, srcbuf, adbuf, numbuf, wbuf8,
                 num_acc, denp_acc):
    c = lax.axis_index("c")
    s = lax.axis_index("s")

    _zero_bufs(numbuf, wbuf8)
    row0 = s * ROWS_PER_TILE
    drow0 = s * DENP_PER_TILE

    @pl.loop(0, ROWS_PER_TILE // B)
    def _(i):
        pltpu.sync_copy(numbuf, num_acc.at[pl.ds(row0 + i * B, B)])

    @pl.loop(0, DENP_PER_TILE // 80)
    def _(i):
        pltpu.sync_copy(wbuf8.at[pl.ds(0, 80)],
                        denp_acc.at[pl.ds(drow0 + i * 80, 80)])

    plsc.subcore_barrier()

    adc = jnp.full((16,), c * 2, _i32)

    @pl.when(c == 0)
    def _():
        _main_loop(srcf, ad, srci, dsti, adc, sbuf, dbuf, dbuf4, srcbuf,
                   adbuf, numbuf, wbuf8, num_acc, denp_acc, s)

    @pl.when(c == 1)
    def _():
        _main_loop(srcg, ad, srci, dsti, adc, sbuf, dbuf, dbuf4, srcbuf,
                   adbuf, numbuf, wbuf8, num_acc, denp_acc, s)

    plsc.subcore_barrier()

    @pl.loop(0, ROWS_PER_TILE // B)
    def _(i):
        r = row0 + i * B

        @pl.when(c == 0)
        def _():
            pltpu.sync_copy(num_acc.at[pl.ds(r, B)], numf.at[pl.ds(r, B)])

        @pl.when(c == 1)
        def _():
            pltpu.sync_copy(num_acc.at[pl.ds(r, B)], numg.at[pl.ds(r, B)])

    @pl.loop(0, DENP_PER_TILE // 80)
    def _(i):
        r = drow0 + i * 80

        @pl.when(c == 0)
        def _():
            pltpu.sync_copy(denp_acc.at[pl.ds(r, 80)], denpf.at[pl.ds(r, 80)])

        @pl.when(c == 1)
        def _():
            pltpu.sync_copy(denp_acc.at[pl.ds(r, 80)], denpg.at[pl.ds(r, 80)])


def _edge_phase(srcf, srcg, ad, srci, dsti):
    mesh = plsc.VectorSubcoreMesh(core_axis_name="c", subcore_axis_name="s",
                                  num_cores=2, num_subcores=16)
    f = functools.partial(
        pl.kernel,
        out_type=[
            jax.ShapeDtypeStruct((NPAD_SC, 32), _f32),
            jax.ShapeDtypeStruct((DENP_ROWS, 8), _f32),
            jax.ShapeDtypeStruct((NPAD_SC, 32), _f32),
            jax.ShapeDtypeStruct((DENP_ROWS, 8), _f32),
        ],
        mesh=mesh,
        scratch_types=[
            pltpu.VMEM((B,), _i32),
            pltpu.VMEM((B,), _i32),
            pltpu.VMEM((B,), _i32),
            pltpu.VMEM((B, 48), _f32),
            pltpu.VMEM((B, 8), _f32),
            pltpu.VMEM((B, 32), _f32),
            pltpu.VMEM((B, 8), _f32),
            pltpu.VMEM_SHARED((NPAD_SC, 32), _f32),
            pltpu.VMEM_SHARED((DENP_ROWS, 8), _f32),
        ],
        compiler_params=pltpu.CompilerParams(needs_layout_passes=False,
                                             use_tc_tiling_on_sc=False),
    )(_edge_kernel)
    return f(srcf, srcg, ad, srci, dsti)


def _epi_body(numf_ref, denf_ref, numg_ref, deng_ref, x_ref, b_ref,
              x1n_ref, x2n_ref, ld_ref):
    numf = numf_ref[...]
    numg = numg_ref[...]
    denf = denf_ref[...]
    deng = deng_ref[...]
    xb = x_ref[...]
    b = b_ref[...]
    s1 = numf[:, 0:16] / denf[:, 0:1] + b[0:1, :]
    t1 = numf[:, 16:32] / denf[:, 1:2] + b[1:2, :]
    s2 = numg[:, 0:16] / deng[:, 0:1] + b[2:3, :]
    t2 = numg[:, 16:32] / deng[:, 1:2] + b[3:4, :]
    x1n = xb[:, 16:32] * jnp.exp(s1) + t1
    x2n = x1n * jnp.exp(s2) + t2
    x1n_ref[...] = x1n
    x2n_ref[...] = x2n
    ld_ref[...] = jnp.sum(s1 + s2, axis=1, keepdims=True)


def _epilogue(numf, denf, numg, deng, x, bvec):
    grid = N // BM_EPI
    return pl.pallas_call(
        _epi_body,
        grid=(grid,),
        in_specs=[
            pl.BlockSpec((BM_EPI, 32), lambda i: (i, 0)),
            pl.BlockSpec((BM_EPI, 2), lambda i: (i, 0)),
            pl.BlockSpec((BM_EPI, 32), lambda i: (i, 0)),
            pl.BlockSpec((BM_EPI, 2), lambda i: (i, 0)),
            pl.BlockSpec((BM_EPI, 32), lambda i: (i, 0)),
            pl.BlockSpec((4, 16), lambda i: (0, 0)),
        ],
        out_specs=[
            pl.BlockSpec((BM_EPI, 16), lambda i: (i, 0)),
            pl.BlockSpec((BM_EPI, 16), lambda i: (i, 0)),
            pl.BlockSpec((BM_EPI, 1), lambda i: (i, 0)),
        ],
        out_shape=[
            jax.ShapeDtypeStruct((N, 16), _f32),
            jax.ShapeDtypeStruct((N, 16), _f32),
            jax.ShapeDtypeStruct((N, 1), _f32),
        ],
    )(numf, denf, numg, deng, x, bvec)


def kernel(x, edge_index,
           W_F1, asrc_F1, adst_F1, b_F1,
           W_F2, asrc_F2, adst_F2, b_F2,
           W_G1, asrc_G1, adst_G1, b_G1,
           W_G2, asrc_G2, adst_G2, b_G2):
    w_all = jnp.zeros((32, 64), _f32)
    w_all = w_all.at[0:16, 0:16].set(W_F1)
    w_all = w_all.at[0:16, 16:32].set(W_F2)
    w_all = w_all.at[16:32, 32:48].set(W_G1)
    w_all = w_all.at[16:32, 48:64].set(W_G2)
    p_all = jnp.zeros((64, 8), _f32)
    p_all = p_all.at[0:16, 0].set(asrc_F1)
    p_all = p_all.at[16:32, 1].set(asrc_F2)
    p_all = p_all.at[32:48, 2].set(asrc_G1)
    p_all = p_all.at[48:64, 3].set(asrc_G2)
    p_all = p_all.at[0:16, 4].set(adst_F1)
    p_all = p_all.at[16:32, 5].set(adst_F2)
    p_all = p_all.at[32:48, 6].set(adst_G1)
    p_all = p_all.at[48:64, 7].set(adst_G2)

    x_pad = jnp.pad(x, ((0, NPAD_DENSE - N), (0, 0)))
    loops = jnp.arange(N, dtype=_i32)
    pad_idx = jnp.full((E_PAD - E_TOT,), N, _i32)
    srci = jnp.concatenate([edge_index[0].astype(_i32), loops, pad_idx])
    dsti = jnp.concatenate([edge_index[1].astype(_i32), loops, pad_idx])

    srcf, srcg, ad = _dense_tables(x_pad, w_all, p_all)
    numf, denpf, numg, denpg = _edge_phase(srcf, srcg, ad, srci, dsti)
    denf = denpf.reshape(NPAD_SC, 2)
    deng = denpg.reshape(NPAD_SC, 2)

    bvec = jnp.stack([b_F1, b_F2, b_G1, b_G2])
    x1n, x2n, ld = _epilogue(numf, denf, numg, deng, x, bvec)
    return (x1n, x2n, ld[:, 0])


# speedup vs baseline: 39.7845x; 1.1723x over previous
import functools

import jax
import jax.numpy as jnp
from jax import lax
from jax.experimental import pallas as pl
from jax.experimental.pallas import tpu as pltpu
from jax.experimental.pallas import tpu_sc as plsc

N = 50000
E = 1600000
D = 16

NPAD_DENSE = 50176
BM_DENSE = 512
NPAD_SC = 51200
ROWS_PER_TILE = NPAD_SC // 16
B = 128
E_TOT = E + N
CHUNKS_PER_TILE = -(-E_TOT // (16 * B))
E_PAD = 16 * B * CHUNKS_PER_TILE
DENP_ROWS = NPAD_SC // 4
DENP_PER_TILE = DENP_ROWS // 16
BM_EPI = 400

_i32 = jnp.int32
_f32 = jnp.float32


def _dense_body(x_ref, w_ref, p_ref, srcf_ref, srcg_ref, ad_ref):
    x = x_ref[...]
    h = jnp.dot(x, w_ref[...], preferred_element_type=_f32)
    a = jnp.dot(h, p_ref[...], preferred_element_type=_f32)
    zpad = jnp.zeros((x.shape[0], 14), _f32)
    srcf_ref[...] = jnp.concatenate([h[:, 0:32], a[:, 0:2], zpad], axis=1)
    srcg_ref[...] = jnp.concatenate([h[:, 32:64], a[:, 2:4], zpad], axis=1)
    ad_ref[...] = jnp.concatenate([a[:, 4:8], jnp.zeros((x.shape[0], 4), _f32)],
                                  axis=1)


def _dense_tables(x_pad, w_all, p_all):
    grid = NPAD_DENSE // BM_DENSE
    return pl.pallas_call(
        _dense_body,
        grid=(grid,),
        in_specs=[
            pl.BlockSpec((BM_DENSE, 32), lambda i: (i, 0)),
            pl.BlockSpec((32, 64), lambda i: (0, 0)),
            pl.BlockSpec((64, 8), lambda i: (0, 0)),
        ],
        out_specs=[
            pl.BlockSpec((BM_DENSE, 48), lambda i: (i, 0)),
            pl.BlockSpec((BM_DENSE, 48), lambda i: (i, 0)),
            pl.BlockSpec((BM_DENSE, 8), lambda i: (i, 0)),
        ],
        out_shape=[
            jax.ShapeDtypeStruct((NPAD_DENSE, 48), _f32),
            jax.ShapeDtypeStruct((NPAD_DENSE, 48), _f32),
            jax.ShapeDtypeStruct((NPAD_DENSE, 8), _f32),
        ],
    )(x_pad, w_all, p_all)


def _zero_bufs(numbuf, wbuf8):
    zv = jnp.zeros((16,), _f32)
    cols = lax.iota(_i32, 16)

    @pl.loop(0, B)
    def _(e):
        row = jnp.full((16,), e, _i32)
        plsc.store_scatter(numbuf, [row, cols], zv)
        plsc.store_scatter(numbuf, [row, cols + 16], zv)

    @pl.loop(0, B // 2)
    def _(e):
        row = e * 2 + (cols >> 3)
        plsc.store_scatter(wbuf8, [row, cols & 7], zv)


def _main_loop(tbl_hbm, ad_hbm, srci_hbm, dsti_hbm, adc,
               sbuf, dbuf, dbuf4, srcbuf, adbuf, numbuf, wbuf8,
               sem_a, sem_b, num_acc, denp_acc, s):
    cols = lax.iota(_i32, 16)
    zv = jnp.zeros((16,), _f32)

    @pl.loop(0, CHUNKS_PER_TILE)
    def _(g):
        base = (s * CHUNKS_PER_TILE + g) * B
        ci = pltpu.async_copy(srci_hbm.at[pl.ds(base, B)], sbuf, sem_a)
        cd = pltpu.async_copy(dsti_hbm.at[pl.ds(base, B)], dbuf, sem_b)
        ci.wait()
        cd.wait()
        gs = pltpu.async_copy(tbl_hbm.at[sbuf], srcbuf, sem_a)
        ga = pltpu.async_copy(ad_hbm.at[dbuf], adbuf, sem_b)
        gs.wait()
        ga.wait()

        for j in range(B // 16):
            rows = cols + j * 16
            as0 = plsc.load_gather(srcbuf, [rows, jnp.full((16,), 32, _i32)])
            as1 = plsc.load_gather(srcbuf, [rows, jnp.full((16,), 33, _i32)])
            ad0 = plsc.load_gather(adbuf, [rows, adc])
            ad1 = plsc.load_gather(adbuf, [rows, adc + 1])
            al0 = as0 + ad0
            al1 = as1 + ad1
            w0 = jnp.exp(jnp.where(al0 >= 0.0, al0, al0 * 0.2))
            w1 = jnp.exp(jnp.where(al1 >= 0.0, al1, al1 * 0.2))
            d = plsc.load_gather(dbuf, [rows])
            plsc.store_scatter(dbuf4, [rows], d >> 2)
            for k in range(8):
                plsc.store_scatter(wbuf8, [rows, jnp.full((16,), k, _i32)], zv)
            dcol = (d & 3) * 2
            plsc.store_scatter(wbuf8, [rows, dcol], w0)
            plsc.store_scatter(wbuf8, [rows, dcol + 1], w1)
            for k in range(16):
                ck = jnp.full((16,), k, _i32)
                hk = plsc.load_gather(srcbuf, [rows, ck])
                plsc.store_scatter(numbuf, [rows, ck], hk * w0)
            for k in range(16, 32):
                ck = jnp.full((16,), k, _i32)
                hk = plsc.load_gather(srcbuf, [rows, ck])
                plsc.store_scatter(numbuf, [rows, ck], hk * w1)

        sn = pltpu.async_copy(numbuf, num_acc.at[dbuf], sem_a, add=True)
        sw = pltpu.async_copy(wbuf8, denp_acc.at[dbuf4], sem_b, add=True)
        sn.wait()
        sw.wait()


def _edge_kernel(srcf, srcg, ad, srci, dsti,
                 numf, denpf, numg, denpg,
                 sbuf, dbuf, dbuf4, srcbuf, adbuf, numbuf, wbuf8,
                 sem_a, sem_b, num_acc, denp_acc):
    c = lax.axis_index("c")
    s = lax.axis_index("s")

    _zero_bufs(numbuf, wbuf8)
    row0 = s * ROWS_PER_TILE
    drow0 = s * DENP_PER_TILE

    @pl.loop(0, ROWS_PER_TILE // B)
    def _(i):
        pltpu.sync_copy(numbuf, num_acc.at[pl.ds(row0 + i * B, B)])

    @pl.loop(0, DENP_PER_TILE // 80)
    def _(i):
        pltpu.sync_copy(wbuf8.at[pl.ds(0, 80)],
                        denp_acc.at[pl.ds(drow0 + i * 80, 80)])

    plsc.subcore_barrier()

    adc = jnp.full((16,), c * 2, _i32)

    @pl.when(c == 0)
    def _():
        _main_loop(srcf, ad, srci, dsti, adc, sbuf, dbuf, dbuf4, srcbuf,
                   adbuf, numbuf, wbuf8, sem_a, sem_b, num_acc, denp_acc, s)

    @pl.when(c == 1)
    def _():
        _main_loop(srcg, ad, srci, dsti, adc, sbuf, dbuf, dbuf4, srcbuf,
                   adbuf, numbuf, wbuf8, sem_a, sem_b, num_acc, denp_acc, s)

    plsc.subcore_barrier()

    @pl.loop(0, ROWS_PER_TILE // B)
    def _(i):
        r = row0 + i * B

        @pl.when(c == 0)
        def _():
            pltpu.sync_copy(num_acc.at[pl.ds(r, B)], numf.at[pl.ds(r, B)])

        @pl.when(c == 1)
        def _():
            pltpu.sync_copy(num_acc.at[pl.ds(r, B)], numg.at[pl.ds(r, B)])

    @pl.loop(0, DENP_PER_TILE // 80)
    def _(i):
        r = drow0 + i * 80

        @pl.when(c == 0)
        def _():
            pltpu.sync_copy(denp_acc.at[pl.ds(r, 80)], denpf.at[pl.ds(r, 80)])

        @pl.when(c == 1)
        def _():
            pltpu.sync_copy(denp_acc.at[pl.ds(r, 80)], denpg.at[pl.ds(r, 80)])


def _edge_phase(srcf, srcg, ad, srci, dsti):
    mesh = plsc.VectorSubcoreMesh(core_axis_name="c", subcore_axis_name="s",
                                  num_cores=2, num_subcores=16)
    f = functools.partial(
        pl.kernel,
        out_type=[
            jax.ShapeDtypeStruct((NPAD_SC, 32), _f32),
            jax.ShapeDtypeStruct((DENP_ROWS, 8), _f32),
            jax.ShapeDtypeStruct((NPAD_SC, 32), _f32),
            jax.ShapeDtypeStruct((DENP_ROWS, 8), _f32),
        ],
        mesh=mesh,
        scratch_types=[
            pltpu.VMEM((B,), _i32),
            pltpu.VMEM((B,), _i32),
            pltpu.VMEM((B,), _i32),
            pltpu.VMEM((B, 48), _f32),
            pltpu.VMEM((B, 8), _f32),
            pltpu.VMEM((B, 32), _f32),
            pltpu.VMEM((B, 8), _f32),
            pltpu.SemaphoreType.DMA,
            pltpu.SemaphoreType.DMA,
            pltpu.VMEM_SHARED((NPAD_SC, 32), _f32),
            pltpu.VMEM_SHARED((DENP_ROWS, 8), _f32),
        ],
        compiler_params=pltpu.CompilerParams(needs_layout_passes=False,
                                             use_tc_tiling_on_sc=False),
    )(_edge_kernel)
    return f(srcf, srcg, ad, srci, dsti)


def _epi_body(numf_ref, denf_ref, numg_ref, deng_ref, x_ref, b_ref,
              x1n_ref, x2n_ref, ld_ref):
    numf = numf_ref[...]
    numg = numg_ref[...]
    denf = denf_ref[...]
    deng = deng_ref[...]
    xb = x_ref[...]
    b = b_ref[...]
    s1 = numf[:, 0:16] / denf[:, 0:1] + b[0:1, :]
    t1 = numf[:, 16:32] / denf[:, 1:2] + b[1:2, :]
    s2 = numg[:, 0:16] / deng[:, 0:1] + b[2:3, :]
    t2 = numg[:, 16:32] / deng[:, 1:2] + b[3:4, :]
    x1n = xb[:, 16:32] * jnp.exp(s1) + t1
    x2n = x1n * jnp.exp(s2) + t2
    x1n_ref[...] = x1n
    x2n_ref[...] = x2n
    ld_ref[...] = jnp.sum(s1 + s2, axis=1, keepdims=True)


def _epilogue(numf, denf, numg, deng, x, bvec):
    grid = N // BM_EPI
    return pl.pallas_call(
        _epi_body,
        grid=(grid,),
        in_specs=[
            pl.BlockSpec((BM_EPI, 32), lambda i: (i, 0)),
            pl.BlockSpec((BM_EPI, 2), lambda i: (i, 0)),
            pl.BlockSpec((BM_EPI, 32), lambda i: (i, 0)),
            pl.BlockSpec((BM_EPI, 2), lambda i: (i, 0)),
            pl.BlockSpec((BM_EPI, 32), lambda i: (i, 0)),
            pl.BlockSpec((4, 16), lambda i: (0, 0)),
        ],
        out_specs=[
            pl.BlockSpec((BM_EPI, 16), lambda i: (i, 0)),
            pl.BlockSpec((BM_EPI, 16), lambda i: (i, 0)),
            pl.BlockSpec((BM_EPI, 1), lambda i: (i, 0)),
        ],
        out_shape=[
            jax.ShapeDtypeStruct((N, 16), _f32),
            jax.ShapeDtypeStruct((N, 16), _f32),
            jax.ShapeDtypeStruct((N, 1), _f32),
        ],
    )(numf, denf, numg, deng, x, bvec)


def kernel(x, edge_index,
           W_F1, asrc_F1, adst_F1, b_F1,
           W_F2, asrc_F2, adst_F2, b_F2,
           W_G1, asrc_G1, adst_G1, b_G1,
           W_G2, asrc_G2, adst_G2, b_G2):
    w_all = jnp.zeros((32, 64), _f32)
    w_all = w_all.at[0:16, 0:16].set(W_F1)
    w_all = w_all.at[0:16, 16:32].set(W_F2)
    w_all = w_all.at[16:32, 32:48].set(W_G1)
    w_all = w_all.at[16:32, 48:64].set(W_G2)
    p_all = jnp.zeros((64, 8), _f32)
    p_all = p_all.at[0:16, 0].set(asrc_F1)
    p_all = p_all.at[16:32, 1].set(asrc_F2)
    p_all = p_all.at[32:48, 2].set(asrc_G1)
    p_all = p_all.at[48:64, 3].set(asrc_G2)
    p_all = p_all.at[0:16, 4].set(adst_F1)
    p_all = p_all.at[16:32, 5].set(adst_F2)
    p_all = p_all.at[32:48, 6].set(adst_G1)
    p_all = p_all.at[48:64, 7].set(adst_G2)

    x_pad = jnp.pad(x, ((0, NPAD_DENSE - N), (0, 0)))
    loops = jnp.arange(N, dtype=_i32)
    pad_idx = jnp.full((E_PAD - E_TOT,), N, _i32)
    srci = jnp.concatenate([edge_index[0].astype(_i32), loops, pad_idx])
    dsti = jnp.concatenate([edge_index[1].astype(_i32), loops, pad_idx])

    srcf, srcg, ad = _dense_tables(x_pad, w_all, p_all)
    numf, denpf, numg, denpg = _edge_phase(srcf, srcg, ad, srci, dsti)
    denf = denpf.reshape(NPAD_SC, 2)
    deng = denpg.reshape(NPAD_SC, 2)

    bvec = jnp.stack([b_F1, b_F2, b_G1, b_G2])
    x1n, x2n, ld = _epilogue(numf, denf, numg, deng, x, bvec)
    return (x1n, x2n, ld[:, 0])
